# baseline (speedup 1.0000x reference)
import functools

import jax
import jax.numpy as jnp
from jax import lax
from jax.experimental import pallas as pl
from jax.experimental.pallas import tpu as pltpu
from jax.experimental.pallas import tpu_sc as plsc

_VOCAB = 1000000
_D = 64
_B = 4096
_L = 200
_NW = 32
_PER_W = _B // _NW
_SCALE = float(_D) ** 0.5
_VECS = _D // 16


def _emb_body(x_hbm, table_hbm, out_hbm, idx_v, rows_v, sem):
    wid = lax.axis_index("s") * 2 + lax.axis_index("c")
    base = wid * _PER_W

    def row_body(r, _):
        b = base + r
        pltpu.sync_copy(x_hbm.at[b], idx_v)
        pltpu.async_copy(table_hbm.at[idx_v], rows_v, sem).wait()

        def scale_body(i, _):
            for j in range(_VECS):
                sl = (i, pl.ds(j * 16, 16))
                rows_v[sl] = rows_v[sl] * _SCALE
            return 0

        lax.fori_loop(0, _L, scale_body, 0)
        pltpu.sync_copy(rows_v, out_hbm.at[b])
        return 0

    lax.fori_loop(0, _PER_W, row_body, 0)


_emb = functools.partial(
    pl.kernel,
    out_type=jax.ShapeDtypeStruct((_B, _L, _D), jnp.float32),
    mesh=plsc.VectorSubcoreMesh(core_axis_name="c", subcore_axis_name="s"),
    scratch_types=[
        pltpu.VMEM((_L,), jnp.int32),
        pltpu.VMEM((_L, _D), jnp.float32),
        pltpu.SemaphoreType.DMA,
    ],
    compiler_params=pltpu.CompilerParams(use_tc_tiling_on_sc=False),
)(_emb_body)


@jax.jit
def kernel(x, emb_weight):
    return _emb(x, emb_weight)


# speedup vs baseline: 1.1788x; 1.1788x over previous
import functools

import jax
import jax.numpy as jnp
from jax import lax
from jax.experimental import pallas as pl
from jax.experimental.pallas import tpu as pltpu
from jax.experimental.pallas import tpu_sc as plsc

_VOCAB = 1000000
_D = 64
_B = 4096
_L = 200
_SCALE = float(_D) ** 0.5


def _emb_body(xT_hbm, tbl_hbm, out_hbm, raws, idxs, prows, rows, st,
              isem, gsem, osem):
    wid = lax.axis_index("s") * 2 + lax.axis_index("c")
    bbase = wid * 128

    def issue_raw(l, slot):
        pltpu.async_copy(xT_hbm.at[l, pl.ds(bbase, 128)], raws.at[slot], isem)

    def prep_and_gather(l, slot):
        pltpu.make_async_copy(
            xT_hbm.at[l, pl.ds(bbase, 128)], raws.at[slot], isem).wait()
        for j in range(8):
            v = raws[slot, pl.ds(16 * j, 16)]
            idxs[slot, pl.ds(16 * j, 16)] = v >> 1
            prows[slot, pl.ds(16 * j, 16)] = (v & 1) * 64
        pltpu.async_copy(tbl_hbm.at[idxs.at[slot]], rows.at[slot], gsem)

    def scale(slot, sslot):
        def _group(bg, _):
            off_v = prows[slot, pl.ds(bg * 16, 16)]
            for k in range(16):
                bm = bg * 16 + k
                off = off_v[k]
                for j in range(4):
                    st[sslot, bm, pl.ds(16 * j, 16)] = (
                        rows[slot, bm, pl.ds(off + 16 * j, 16)] * _SCALE
                    )
            return 0

        lax.fori_loop(0, 8, _group, 0)

    for l in range(4):
        issue_raw(l, l)
    for l in range(3):
        prep_and_gather(l, l)

    def l_body(lp, _):
        for p in range(4):
            l = 4 * lp + p
            k3 = (p + 3) % 4
            sslot = p % 2

            @pl.when(l + 3 < _L)
            def _():
                prep_and_gather(l + 3, k3)

            @pl.when(l + 4 < _L)
            def _():
                issue_raw(l + 4, p)

            pltpu.make_async_copy(
                tbl_hbm.at[idxs.at[p]], rows.at[p], gsem).wait()

            @pl.when(l >= 2)
            def _():
                pltpu.make_async_copy(
                    st.at[sslot], out_hbm.at[l - 2, pl.ds(bbase, 128)],
                    osem).wait()

            scale(p, sslot)
            pltpu.async_copy(
                st.at[sslot], out_hbm.at[l, pl.ds(bbase, 128)], osem)
        return 0

    lax.fori_loop(0, _L // 4, l_body, 0)

    pltpu.make_async_copy(
        st.at[0], out_hbm.at[_L - 2, pl.ds(bbase, 128)], osem).wait()
    pltpu.make_async_copy(
        st.at[1], out_hbm.at[_L - 1, pl.ds(bbase, 128)], osem).wait()


_emb = functools.partial(
    pl.kernel,
    out_type=jax.ShapeDtypeStruct((_L, _B, _D), jnp.float32),
    mesh=plsc.VectorSubcoreMesh(core_axis_name="c", subcore_axis_name="s"),
    scratch_types=[
        pltpu.VMEM((4, 128), jnp.int32),
        pltpu.VMEM((4, 128), jnp.int32),
        pltpu.VMEM((4, 128), jnp.int32),
        pltpu.VMEM((4, 128, 128), jnp.float32),
        pltpu.VMEM((2, 128, _D), jnp.float32),
        pltpu.SemaphoreType.DMA,
        pltpu.SemaphoreType.DMA,
        pltpu.SemaphoreType.DMA,
    ],
    compiler_params=pltpu.CompilerParams(use_tc_tiling_on_sc=True),
)(_emb_body)


@jax.jit
def kernel(x, emb_weight):
    xT = x.T
    tbl = emb_weight.reshape(_VOCAB // 2, 128)
    inter = _emb(xT, tbl)
    return inter.transpose(1, 0, 2)


# speedup vs baseline: 1.4230x; 1.2072x over previous
import functools

import jax
import jax.numpy as jnp
from jax import lax
from jax.experimental import pallas as pl
from jax.experimental.pallas import tpu as pltpu
from jax.experimental.pallas import tpu_sc as plsc

_VOCAB = 1000000
_D = 64
_B = 4096
_L = 200
_SCALE = float(_D) ** 0.5


def _emb_body(xT_hbm, tbl_hbm, out_hbm, raws, idxs, prows, rows, st,
              isem, gsem, osem):
    wid = lax.axis_index("s") * 2 + lax.axis_index("c")
    bbase = wid * 128

    def issue_raw(l, slot):
        pltpu.async_copy(xT_hbm.at[l, pl.ds(bbase, 128)], raws.at[slot], isem)

    def prep_and_gather(l, slot):
        pltpu.make_async_copy(
            xT_hbm.at[l, pl.ds(bbase, 128)], raws.at[slot], isem).wait()
        for j in range(8):
            v = raws[slot, pl.ds(16 * j, 16)]
            idxs[slot, pl.ds(16 * j, 16)] = v >> 1
            prows[slot, pl.ds(16 * j, 16)] = (v & 1) * 64
        pltpu.async_copy(tbl_hbm.at[idxs.at[slot]], rows.at[slot], gsem)

    def scale(slot, sslot):
        def _group(bg, _):
            w_hi = prows[slot, pl.ds(bg * 16, 16)].astype(jnp.float32) * (
                _SCALE / 64.0)
            for k in range(16):
                bm = bg * 16 + k
                whi = w_hi[k]
                wlo = _SCALE - whi
                for j in range(4):
                    lo = rows[slot, bm, pl.ds(16 * j, 16)]
                    hi = rows[slot, bm, pl.ds(64 + 16 * j, 16)]
                    st[sslot, bm, pl.ds(16 * j, 16)] = lo * wlo + hi * whi
            return 0

        lax.fori_loop(0, 8, _group, 0)

    for l in range(4):
        issue_raw(l, l)
    for l in range(3):
        prep_and_gather(l, l)

    def l_body(lp, _):
        for p in range(4):
            l = 4 * lp + p
            k3 = (p + 3) % 4
            sslot = p % 2

            @pl.when(l + 3 < _L)
            def _():
                prep_and_gather(l + 3, k3)

            @pl.when(l + 4 < _L)
            def _():
                issue_raw(l + 4, p)

            pltpu.make_async_copy(
                tbl_hbm.at[idxs.at[p]], rows.at[p], gsem).wait()

            @pl.when(l >= 2)
            def _():
                pltpu.make_async_copy(
                    st.at[sslot], out_hbm.at[l - 2, pl.ds(bbase, 128)],
                    osem).wait()

            scale(p, sslot)
            pltpu.async_copy(
                st.at[sslot], out_hbm.at[l, pl.ds(bbase, 128)], osem)
        return 0

    lax.fori_loop(0, _L // 4, l_body, 0)

    pltpu.make_async_copy(
        st.at[0], out_hbm.at[_L - 2, pl.ds(bbase, 128)], osem).wait()
    pltpu.make_async_copy(
        st.at[1], out_hbm.at[_L - 1, pl.ds(bbase, 128)], osem).wait()


_emb = functools.partial(
    pl.kernel,
    out_type=jax.ShapeDtypeStruct((_L, _B, _D), jnp.float32),
    mesh=plsc.VectorSubcoreMesh(core_axis_name="c", subcore_axis_name="s"),
    scratch_types=[
        pltpu.VMEM((4, 128), jnp.int32),
        pltpu.VMEM((4, 128), jnp.int32),
        pltpu.VMEM((4, 128), jnp.int32),
        pltpu.VMEM((4, 128, 128), jnp.float32),
        pltpu.VMEM((2, 128, _D), jnp.float32),
        pltpu.SemaphoreType.DMA,
        pltpu.SemaphoreType.DMA,
        pltpu.SemaphoreType.DMA,
    ],
    compiler_params=pltpu.CompilerParams(use_tc_tiling_on_sc=True),
)(_emb_body)


@jax.jit
def kernel(x, emb_weight):
    xT = x.T
    tbl = emb_weight.reshape(_VOCAB // 2, 128)
    inter = _emb(xT, tbl)
    return inter.transpose(1, 0, 2)
